# baseline (speedup 1.0000x reference)
import functools

import jax
import jax.numpy as jnp
from jax import lax
from jax.experimental import pallas as pl
from jax.experimental.pallas import tpu as pltpu
from jax.experimental.pallas import tpu_sc as plsc

N_ATOMS = 500000
SPECIES = 10
NSEG = 16
LANES = 16
NW = 32

AFLAT = N_ATOMS * SPECIES
PFLAT = N_ATOMS * 3
CA = 20000
CP = 12000
NGA = AFLAT // CA
NGP = PFLAT // CP
GPTA = -(-NGA // NW)
GPTP = -(-NGP // NW)


def _phase(src_hbm, bufs, sems, cut_ref, wvs, cdim, n_chunks, per_worker,
           cut_acc, acc_base, wid):
    groups = cdim // (5 * LANES)
    g_lo = wid * per_worker
    g_hi = jnp.minimum(g_lo + per_worker, n_chunks)
    cvec = cut_ref[...]
    cks = [cvec[k] for k in range(NSEG)]
    iota = lax.iota(jnp.int32, LANES)
    zero_v = jnp.zeros((LANES,), jnp.float32)

    def weighted(x, u):
        if wvs is None:
            return x * x
        return x * wvs[u]

    def issue(g, b):
        pltpu.async_copy(src_hbm.at[pl.ds(g * cdim, cdim)], bufs[b],
                         sems[b])

    def wait(b):
        pltpu.make_async_copy(src_hbm.at[pl.ds(0, cdim)], bufs[b],
                              sems[b]).wait()

    def chunk_total(bufb):
        def body(i, accs):
            base = i * (5 * LANES)
            return tuple(accs[u] + weighted(bufb[pl.ds(base + u * LANES,
                                                       LANES)], u)
                         for u in range(5))
        accs = lax.fori_loop(0, groups, body, (zero_v,) * 5)
        return jnp.sum(accs[0] + accs[1] + accs[2] + accs[3] + accs[4])

    def masked_partial(bufb, lo, ck):
        def body(i, accs):
            base = i * (5 * LANES)
            out = []
            for u in range(5):
                off = base + u * LANES
                x = weighted(bufb[pl.ds(off, LANES)], u)
                f = lo + off + iota
                out.append(accs[u] + jnp.where(f < ck, x, 0.0))
            return tuple(out)
        accs = lax.fori_loop(0, groups, body, (zero_v,) * 5)
        return jnp.sum(accs[0] + accs[1] + accs[2] + accs[3] + accs[4])

    for b in range(2):
        @pl.when(g_lo + b < g_hi)
        def _():
            issue(g_lo + b, b)

    def outer(t, carry):
        run, cutvec = carry
        for b in range(2):
            g = g_lo + 2 * t + b
            active = g < g_hi
            lo = g * cdim
            hi = lo + cdim

            @pl.when(active)
            def _():
                wait(b)
            bufb = bufs[b]
            total = chunk_total(bufb)

            for k in range(NSEG):
                def hit(bufb=bufb, lo=lo, k=k, run=run, cutvec=cutvec):
                    part = masked_partial(bufb, lo, cks[k])
                    return jnp.where(iota == k, run + part, cutvec)

                def miss(cutvec=cutvec):
                    return cutvec

                straddle = jnp.logical_and(
                    active,
                    jnp.logical_and(cks[k] >= lo, cks[k] < hi))
                cutvec = lax.cond(straddle, hit, miss)

            @pl.when(jnp.logical_and(active, g + 2 < g_hi))
            def _():
                issue(g + 2, b)
            run = jnp.where(active, run + total, run)
        return run, cutvec

    n_my = g_hi - g_lo
    run_total, cutvec = lax.fori_loop(
        0, (n_my + 1) // 2, outer,
        (jnp.float32(0.0), jnp.zeros((NSEG,), jnp.float32)))

    my_hi = g_hi * cdim
    past = cvec >= my_hi
    cutvec = jnp.where(past, run_total, cutvec)
    cut_acc[pl.ds(acc_base, NSEG)] = cutvec


def _sc_body(attrs_hbm, pos_hbm, c10_hbm, c3_hbm, wpat_hbm, out_hbm,
             abuf0, abuf1, pbuf0, pbuf1, cva, cvp, wv, cut_acc, sem0, sem1):
    wid = lax.axis_index("c") * 16 + lax.axis_index("s")
    pltpu.sync_copy(c10_hbm, cva)
    pltpu.sync_copy(c3_hbm, cvp)
    pltpu.sync_copy(wpat_hbm, wv)
    wvs = [wv[pl.ds(u * LANES, LANES)] for u in range(5)]
    sems = (sem0, sem1)
    _phase(attrs_hbm, (abuf0, abuf1), sems, cva, wvs, CA, NGA, GPTA,
           cut_acc, 0, wid)
    _phase(pos_hbm, (pbuf0, pbuf1), sems, cvp, None, CP, NGP, GPTP,
           cut_acc, NSEG, wid)
    pltpu.sync_copy(cut_acc, out_hbm.at[wid])


_mace_sc = functools.partial(
    pl.kernel,
    out_type=jax.ShapeDtypeStruct((NW, 2 * NSEG), jnp.float32),
    mesh=plsc.VectorSubcoreMesh(core_axis_name="c", subcore_axis_name="s"),
    scratch_types=[
        pltpu.VMEM((CA,), jnp.float32),
        pltpu.VMEM((CA,), jnp.float32),
        pltpu.VMEM((CP,), jnp.float32),
        pltpu.VMEM((CP,), jnp.float32),
        pltpu.VMEM((NSEG,), jnp.int32),
        pltpu.VMEM((NSEG,), jnp.int32),
        pltpu.VMEM((80,), jnp.float32),
        pltpu.VMEM((2 * NSEG,), jnp.float32),
        pltpu.SemaphoreType.DMA,
        pltpu.SemaphoreType.DMA,
    ],
    compiler_params=pltpu.CompilerParams(needs_layout_passes=False),
)(_sc_body)


def kernel(node_attrs, positions, ptr, W, b):
    attrs_flat = node_attrs.reshape(-1)
    pos_flat = positions.reshape(-1)
    ptr = ptr.astype(jnp.int32)
    ends = ptr[1:]
    c10 = ends * SPECIES
    c3 = ends * 3
    wpat = jnp.tile(W.reshape(-1), 8)

    partials = _mace_sc(attrs_flat, pos_flat, c10, c3, wpat)
    colsum = jnp.sum(partials, axis=0)
    cum_a = colsum[:NSEG]
    cum_p = colsum[NSEG:]
    z1 = jnp.zeros((1,), jnp.float32)
    seg_a = cum_a - jnp.concatenate([z1, cum_a[:-1]])
    seg_p = cum_p - jnp.concatenate([z1, cum_p[:-1]])
    counts = (ptr[1:] - ptr[:-1]).astype(jnp.float32)
    return seg_a + 0.5 * seg_p + b[0] * counts
